# baseline (speedup 1.0000x reference)
import functools

import jax
import jax.numpy as jnp
from jax import lax
from jax.experimental import pallas as pl
from jax.experimental.pallas import tpu as pltpu
from jax.experimental.pallas import tpu_sc as plsc

_N = 10000
_E = 320000
_F = 128
_NORM_SCALE = 1.0

_NC = 2
_NS = 16
_NW = _NC * _NS
_EW = _E // _NW
_C = 80
_NCH = 128
_NCHP = 134
_NP = 10240
_RT = _NP // _NS

_mesh = plsc.VectorSubcoreMesh(core_axis_name="c", subcore_axis_name="s")


@functools.partial(
    pl.kernel,
    mesh=_mesh,
    out_type=jax.ShapeDtypeStruct((_NC, _NP, _F), jnp.float32),
    scratch_types=(
        [pltpu.VMEM((_C, _F), jnp.float32) for _ in range(4)]
        + [pltpu.VMEM((3, _C), jnp.int32) for _ in range(8)]
        + [pltpu.VMEM_SHARED((_NP, _F), jnp.float32)]
        + [pltpu.SemaphoreType.DMA for _ in range(16)]
    ),
)
def _spmm(h_hbm, ed_hbm, out_hbm, *refs):
    bufs = refs[0:4]
    ibufs = refs[4:12]
    acc_sh = refs[12]
    gsems = refs[13:17]
    ssems = refs[17:21]
    isems = refs[21:29]
    c = lax.axis_index("c")
    s = lax.axis_index("s")
    wid = c * _NS + s

    def issue_i(b, m):
        pltpu.async_copy(ed_hbm.at[wid, m], ibufs[b], isems[b])

    def wait_i(b, m):
        pltpu.make_async_copy(ed_hbm.at[wid, m], ibufs[b], isems[b]).wait()

    def issue_g(b4, b8):
        pltpu.async_copy(h_hbm.at[ibufs[b8].at[0]], bufs[b4], gsems[b4])

    def wait_g(b4, b8):
        pltpu.make_async_copy(h_hbm.at[ibufs[b8].at[0]], bufs[b4],
                              gsems[b4]).wait()

    def issue_s(b4, b8):
        pltpu.async_copy(bufs[b4], acc_sh.at[ibufs[b8].at[1]],
                         ssems[b4], add=True)

    def wait_s(b4, b8):
        pltpu.make_async_copy(bufs[b4], acc_sh.at[ibufs[b8].at[1]],
                              ssems[b4]).wait()

    for m in range(6):
        issue_i(m, m)
    wait_i(0, 0)
    issue_g(0, 0)
    wait_i(1, 1)
    issue_g(1, 1)

    zvec = jnp.zeros((16,), jnp.float32)

    def zrow(r, carry):
        for j in range(_F // 16):
            bufs[2][r, pl.ds(j * 16, 16)] = zvec
        return carry

    lax.fori_loop(0, _C, zrow, 0)
    for k in range(_RT // _C):
        pltpu.sync_copy(bufs[2], acc_sh.at[pl.ds(s * _RT + k * _C, _C)])
    plsc.subcore_barrier()

    def scale(buf, ib):
        def group(g, gcarry):
            v16 = lax.bitcast_convert_type(
                ib[2, pl.ds(g * 16, 16)], jnp.float32)
            for i in range(16):
                vvec = v16[jnp.full((16,), i, jnp.int32)]
                r = g * 16 + i
                for j in range(_F // 16):
                    seg = buf[r, pl.ds(j * 16, 16)]
                    buf[r, pl.ds(j * 16, 16)] = seg * vvec
            return gcarry

        lax.fori_loop(0, _C // 16, group, 0)

    def oct_(i8, carry):
        for k in range(8):
            m = 8 * i8 + k
            wait_g(k % 4, k)
            scale(bufs[k % 4], ibufs[k])
            issue_s(k % 4, k)
            wait_s(k % 4, k)
            issue_i((k + 6) % 8, m + 6)
            wait_i((k + 2) % 8, m + 2)
            issue_g((k + 2) % 4, (k + 2) % 8)
        return carry

    lax.fori_loop(0, _NCH // 8, oct_, 0)
    wait_g(0, 0)
    wait_g(1, 1)
    for m in range(_NCH + 2, _NCH + 6):
        wait_i(m % 8, m)
    plsc.subcore_barrier()

    r0 = s * _RT
    pltpu.sync_copy(acc_sh.at[pl.ds(r0, _RT)], out_hbm.at[c, pl.ds(r0, _RT)])


def _mm_body(x_ref, w_ref, o_ref):
    o_ref[...] = jnp.dot(x_ref[...], w_ref[...],
                         preferred_element_type=jnp.float32)


def _mid_body(p_ref, b_ref, o_ref):
    agg = p_ref[0, :_N] + p_ref[1, :_N] + b_ref[...]
    col_mean = jnp.mean(agg, axis=0, keepdims=True)
    xc = agg - col_mean
    rownorm_mean = jnp.sqrt(1e-06 + jnp.mean(jnp.sum(xc * xc, axis=1)))
    o_ref[...] = jnp.maximum(_NORM_SCALE * xc / rownorm_mean, 0.0)


def _fin_body(p_ref, w_ref, b_ref, o_ref):
    agg = p_ref[0, :_N] + p_ref[1, :_N]
    o_ref[...] = jnp.dot(agg, w_ref[...],
                         preferred_element_type=jnp.float32) + b_ref[...]


def _pack_edges(edge_index, adj_values):
    pad_e = _NCHP * _C - _EW
    dst = edge_index[0].reshape(_NW, _EW)
    src = edge_index[1].reshape(_NW, _EW)
    vals = lax.bitcast_convert_type(adj_values, jnp.int32).reshape(_NW, _EW)
    pad_dst = jnp.broadcast_to(
        _N + (jnp.arange(pad_e, dtype=jnp.int32) % (_NP - _N)), (_NW, pad_e))
    zi = jnp.zeros((_NW, pad_e), jnp.int32)
    dstp = jnp.concatenate([dst, pad_dst], axis=1).reshape(_NW, _NCHP, _C)
    srcp = jnp.concatenate([src, zi], axis=1).reshape(_NW, _NCHP, _C)
    valsp = jnp.concatenate([vals, zi], axis=1).reshape(_NW, _NCHP, _C)
    return jnp.stack([srcp, dstp, valsp], axis=2)


def kernel(x, edge_index, adj_values, weight1, bias1, weight_out, bias_out):
    ed = _pack_edges(edge_index, adj_values)
    h = pl.pallas_call(
        _mm_body,
        out_shape=jax.ShapeDtypeStruct((_N, _F), jnp.float32),
    )(x, weight1)
    p1 = _spmm(h, ed)
    h2 = pl.pallas_call(
        _mid_body,
        out_shape=jax.ShapeDtypeStruct((_N, _F), jnp.float32),
    )(p1, bias1)
    p2 = _spmm(h2, ed)
    out = pl.pallas_call(
        _fin_body,
        out_shape=jax.ShapeDtypeStruct((_N, 64), jnp.float32),
    )(p2, weight_out, bias_out)
    return out


# speedup vs baseline: 1.4029x; 1.4029x over previous
import functools

import jax
import jax.numpy as jnp
from jax import lax
from jax.experimental import pallas as pl
from jax.experimental.pallas import tpu as pltpu
from jax.experimental.pallas import tpu_sc as plsc

_N = 10000
_E = 320000
_F = 128
_NORM_SCALE = 1.0

_NC = 2
_NS = 16
_NW = _NC * _NS
_EW = _E // _NW
_C = 80
_ECP = 10000
_NP = 10240
_RT = _NP // _NS

_mesh = plsc.VectorSubcoreMesh(core_axis_name="c", subcore_axis_name="s")


@functools.partial(
    pl.kernel,
    mesh=_mesh,
    out_type=jax.ShapeDtypeStruct((_NC, _NP, _F), jnp.float32),
    scratch_types=(
        [pltpu.VMEM((_C, _F), jnp.float32),
         pltpu.VMEM((_C, _F), jnp.float32),
         pltpu.VMEM((_C // 2,), jnp.int32),
         pltpu.VMEM((_C // 2,), jnp.int32),
         pltpu.VMEM((_C,), jnp.int32),
         pltpu.VMEM((_C,), jnp.float32)]
        + [pltpu.VMEM_SHARED((_NP, _F), jnp.float32)]
        + [pltpu.SemaphoreType.DMA for _ in range(2)]
    ),
)
def _spmm(h_hbm, srcA_hbm, srcB_hbm, dst_hbm, vals_hbm, out_hbm,
          rows, zeros_v, srcA, srcB, dst_v, vals_v, acc_sh, gsem0, gsem1):
    c = lax.axis_index("c")
    s = lax.axis_index("s")
    wid = c * _NS + s

    zvec = jnp.zeros((16,), jnp.float32)

    def zrow(r, carry):
        for j in range(_F // 16):
            zeros_v[r, pl.ds(j * 16, 16)] = zvec
        return carry

    lax.fori_loop(0, _C, zrow, 0)
    for k in range(_RT // _C):
        pltpu.sync_copy(zeros_v, acc_sh.at[pl.ds(s * _RT + k * _C, _C)])
    plsc.subcore_barrier()

    def scale(buf, ci):
        def group(g, gcarry):
            v16 = vals_v[pl.ds(g * 16, 16)]
            for i in range(16):
                vvec = v16[jnp.full((16,), i, jnp.int32)]
                r = g * 16 + i
                for j in range(_F // 16):
                    seg = buf[r, pl.ds(j * 16, 16)]
                    buf[r, pl.ds(j * 16, 16)] = seg * vvec
            return gcarry

        lax.fori_loop(0, _C // 16, group, 0)

    _H = _C // 2

    def chunk(ci, carry):
        base = wid * _EW + ci * _C
        pltpu.sync_copy(srcA_hbm.at[pl.ds(base, _H)], srcA)
        pltpu.sync_copy(srcB_hbm.at[pl.ds(base + _H, _H)], srcB)
        pltpu.sync_copy(dst_hbm.at[pl.ds(base, _C)], dst_v)
        pltpu.sync_copy(vals_hbm.at[pl.ds(base, _C)], vals_v)
        d0 = pltpu.async_copy(h_hbm.at[srcA], rows.at[pl.ds(0, _H)], gsem0)
        d1 = pltpu.async_copy(h_hbm.at[srcB], rows.at[pl.ds(_H, _H)], gsem1)
        d0.wait()
        d1.wait()
        scale(rows, ci)
        pltpu.sync_copy(rows, acc_sh.at[dst_v], add=True)
        return carry

    lax.fori_loop(0, _EW // _C, chunk, 0)
    plsc.subcore_barrier()

    r0 = s * _RT
    pltpu.sync_copy(acc_sh.at[pl.ds(r0, _RT)], out_hbm.at[c, pl.ds(r0, _RT)])


def _mm_body(x_ref, w_ref, o_ref):
    o_ref[...] = jnp.dot(x_ref[...], w_ref[...],
                         preferred_element_type=jnp.float32)


def _mid_body(p_ref, b_ref, o_ref):
    agg = p_ref[0, :_N] + p_ref[1, :_N] + b_ref[...]
    col_mean = jnp.mean(agg, axis=0, keepdims=True)
    xc = agg - col_mean
    rownorm_mean = jnp.sqrt(1e-06 + jnp.mean(jnp.sum(xc * xc, axis=1)))
    o_ref[...] = jnp.maximum(_NORM_SCALE * xc / rownorm_mean, 0.0)


def _fin_body(p_ref, w_ref, b_ref, o_ref):
    agg = p_ref[0, :_N] + p_ref[1, :_N]
    o_ref[...] = jnp.dot(agg, w_ref[...],
                         preferred_element_type=jnp.float32) + b_ref[...]


def _pack_edges(edge_index, adj_values):
    pad_e = _ECP - _EW
    dst = edge_index[0].reshape(_NW, _EW)
    src = edge_index[1].reshape(_NW, _EW)
    vals = adj_values.reshape(_NW, _EW)
    pad_dst = jnp.broadcast_to(
        _N + (jnp.arange(pad_e, dtype=jnp.int32) % (_NP - _N)), (_NW, pad_e))
    zi = jnp.zeros((_NW, pad_e), jnp.int32)
    zf = jnp.zeros((_NW, pad_e), jnp.float32)
    dstp = jnp.concatenate([dst, pad_dst], axis=1).reshape(_NW * _ECP)
    srcp = jnp.concatenate([src, zi], axis=1).reshape(_NW * _ECP)
    valsp = jnp.concatenate([vals, zf], axis=1).reshape(_NW * _ECP)
    return srcp, dstp, valsp


def kernel(x, edge_index, adj_values, weight1, bias1, weight_out, bias_out):
    srcp, dstp, valsp = _pack_edges(edge_index, adj_values)
    h = pl.pallas_call(
        _mm_body,
        out_shape=jax.ShapeDtypeStruct((_N, _F), jnp.float32),
    )(x, weight1)
    p1 = _spmm(h, srcp, srcp, dstp, valsp)
    h2 = pl.pallas_call(
        _mid_body,
        out_shape=jax.ShapeDtypeStruct((_N, _F), jnp.float32),
    )(p1, bias1)
    p2 = _spmm(h2, srcp, srcp, dstp, valsp)
    out = pl.pallas_call(
        _fin_body,
        out_shape=jax.ShapeDtypeStruct((_N, 64), jnp.float32),
    )(p2, weight_out, bias_out)
    return out
